# baseline (speedup 1.0000x reference)
import functools

import jax
import jax.numpy as jnp
from jax import lax
from jax.experimental import pallas as pl
from jax.experimental.pallas import tpu as pltpu
from jax.experimental.pallas import tpu_sc as plsc

N_E = 8192
C_DIM = 32
BETA = 0.25
N_TOK = 4096

T = 256
K = 1024
NT = N_TOK // T
NK = N_E // K

_DOT_PREC = jax.lax.Precision.DEFAULT


NH = N_E // 2


def _tc_body(z2_ref, e2_ref, z_ref, emb2_ref,
             oh_ref, idx_ref, loss_ref, acc_s, mini_s):
    i = pl.program_id(0)
    j = pl.program_id(1)
    hcol = jax.lax.broadcasted_iota(jnp.int32, (T, NH), 1)

    @pl.when(j == 0)
    def _argmin_and_left():
        z = z_ref[...]
        z2 = z2_ref[...]
        lcol = jax.lax.broadcasted_iota(jnp.int32, (T, K), 1)

        minv = jnp.full((T, 1), jnp.inf, jnp.float32)
        mini = jnp.zeros((T, 1), jnp.int32)
        for k in range(NK):
            m2 = jax.lax.dot_general(
                z, emb2_ref[k * K:(k + 1) * K, :],
                dimension_numbers=(((1,), (1,)), ((), ())),
                preferred_element_type=jnp.float32,
                precision=_DOT_PREC,
            )
            d = (z2 + e2_ref[:, k * K:(k + 1) * K]) - m2
            rowmin = jnp.min(d, axis=1, keepdims=True)
            rowidx = jnp.min(jnp.where(d == rowmin, lcol, K),
                             axis=1, keepdims=True) + k * K
            upd = rowmin < minv
            mini = jnp.where(upd, rowidx, mini)
            minv = jnp.where(upd, rowmin, minv)
        idx_ref[...] = mini
        mini_s[...] = mini
        oh_ref[...] = (hcol == mini).astype(jnp.float32)

        sq = jnp.sum(minv)

        @pl.when(i == 0)
        def _init():
            acc_s[0] = sq

        @pl.when(i > 0)
        def _acc():
            acc_s[0] += sq

        @pl.when(i == NT - 1)
        def _finalize():
            mse = acc_s[0] / jnp.float32(N_TOK * C_DIM)
            loss_ref[...] = jnp.full((1, 1), mse + BETA * mse, jnp.float32)

    @pl.when(j == 1)
    def _right():
        oh_ref[...] = (hcol == (mini_s[...] - NH)).astype(jnp.float32)


def _tc_call(z2, e2, z_flat, emb2, interpret=False):
    return pl.pallas_call(
        _tc_body,
        grid=(NT, 2),
        in_specs=[
            pl.BlockSpec((T, 1), lambda i, j: (i, 0)),
            pl.BlockSpec((1, N_E), lambda i, j: (0, 0)),
            pl.BlockSpec((T, C_DIM), lambda i, j: (i, 0)),
            pl.BlockSpec((N_E, C_DIM), lambda i, j: (0, 0)),
        ],
        out_specs=[
            pl.BlockSpec((T, NH), lambda i, j: (i, j)),
            pl.BlockSpec((T, 1), lambda i, j: (i, 0)),
            pl.BlockSpec((1, 1), lambda i, j: (0, 0)),
        ],
        out_shape=[
            jax.ShapeDtypeStruct((N_TOK, N_E), jnp.float32),
            jax.ShapeDtypeStruct((N_TOK, 1), jnp.int32),
            jax.ShapeDtypeStruct((1, 1), jnp.float32),
        ],
        scratch_shapes=[
            pltpu.SMEM((1,), jnp.float32),
            pltpu.VMEM((T, 1), jnp.int32),
        ],
        interpret=interpret,
    )(z2, e2, z_flat, emb2)


_SC_INFO = plsc.get_sparse_core_info()
_NC = _SC_INFO.num_cores
_NS = _SC_INFO.num_subcores
_NW = _NC * _NS
_BPW = N_TOK // _NW
_HPW = N_E // _NS


def _sc_body(emb_hbm, idx_hbm, zq_hbm, hist_hbm,
             idx_v, rows_v, ones_v, chunk_v, hist_sh, sem):
    c = lax.axis_index("c")
    s = lax.axis_index("s")
    wid = s * _NC + c
    base = wid * _BPW

    pltpu.sync_copy(idx_hbm.at[pl.ds(base, _BPW)], idx_v)
    pltpu.async_copy(emb_hbm.at[idx_v], rows_v, sem).wait()
    pltpu.sync_copy(rows_v, zq_hbm.at[pl.ds(base, _BPW)])

    for t in range(_HPW // 16):
        chunk_v[pl.ds(t * 16, 16)] = jnp.zeros((16,), jnp.float32)
    for t in range(_BPW // 16):
        ones_v[pl.ds(t * 16, 16)] = jnp.ones((16,), jnp.float32)
    pltpu.sync_copy(chunk_v, hist_sh.at[pl.ds(s * _HPW, _HPW)])
    plsc.subcore_barrier()
    pltpu.sync_copy(ones_v, hist_sh.at[idx_v], add=True)
    plsc.subcore_barrier()
    pltpu.sync_copy(hist_sh.at[pl.ds(s * _HPW, _HPW)], chunk_v)
    pltpu.sync_copy(chunk_v, hist_hbm.at[c, pl.ds(s * _HPW, _HPW)])


@functools.partial(
    pl.kernel,
    mesh=plsc.VectorSubcoreMesh(core_axis_name="c", subcore_axis_name="s"),
    out_type=[
        jax.ShapeDtypeStruct((N_TOK, 128), jnp.float32),
        jax.ShapeDtypeStruct((_NC, N_E), jnp.float32),
    ],
    scratch_types=[
        pltpu.VMEM((_BPW,), jnp.int32),
        pltpu.VMEM((_BPW, 128), jnp.float32),
        pltpu.VMEM((_BPW,), jnp.float32),
        pltpu.VMEM((_HPW,), jnp.float32),
        pltpu.VMEM_SHARED((N_E,), jnp.float32),
        pltpu.SemaphoreType.DMA,
    ],
)
def _sc_call(emb_hbm, idx_hbm, zq_hbm, hist_hbm,
             idx_v, rows_v, ones_v, chunk_v, hist_sh, sem):
    _sc_body(emb_hbm, idx_hbm, zq_hbm, hist_hbm,
             idx_v, rows_v, ones_v, chunk_v, hist_sh, sem)


def _perp_body(hist_ref, perp_ref):
    h = hist_ref[...]
    e_mean = (h[0:1, :] + h[1:2, :]) / jnp.float32(N_TOK)
    ent = -jnp.sum(e_mean * jnp.log(e_mean + 1e-10))
    perp_ref[...] = jnp.full((1, 1), jnp.exp(ent), jnp.float32)


def _perp_call(hist, interpret=False):
    return pl.pallas_call(
        _perp_body,
        grid=(1,),
        in_specs=[pl.BlockSpec((_NC, N_E), lambda i: (0, 0))],
        out_specs=pl.BlockSpec((1, 1), lambda i: (0, 0)),
        out_shape=jax.ShapeDtypeStruct((1, 1), jnp.float32),
        interpret=interpret,
    )(hist)


def kernel(z, emb_weight, *, interpret=False):
    zp = jnp.transpose(z, (0, 2, 3, 1))
    z_flat = zp.reshape(-1, C_DIM)
    z2 = jnp.sum(z_flat ** 2, axis=1, keepdims=True)
    e2 = jnp.sum(emb_weight ** 2, axis=1)[None, :]
    emb2 = emb_weight * 2.0

    min_encodings, min_idx, loss = _tc_call(
        z2, e2, z_flat, emb2, interpret=interpret)

    emb_pad = jnp.pad(emb_weight, ((0, 0), (0, 128 - C_DIM)))
    zq_pad, hist = _sc_call(emb_pad, min_idx.reshape(-1))
    zq = zq_pad[:, :C_DIM]

    perp = _perp_call(hist, interpret=interpret)

    z_q_out = jnp.transpose(zq.reshape(zp.shape), (0, 3, 1, 2))
    return (loss[0, 0], z_q_out, perp[0, 0], min_encodings, min_idx)


# speedup vs baseline: 1.2378x; 1.2378x over previous
import functools

import jax
import jax.numpy as jnp
from jax import lax
from jax.experimental import pallas as pl
from jax.experimental.pallas import tpu as pltpu
from jax.experimental.pallas import tpu_sc as plsc

N_E = 8192
C_DIM = 32
BETA = 0.25
N_TOK = 4096

T = 512
K = 1024
NT = N_TOK // T
NK = N_E // K

_DOT_PREC = jax.lax.Precision.DEFAULT


NH = N_E // 2


def _tc_body(z2_ref, e2_ref, z_ref, emb2_ref,
             oh_ref, idx_ref, loss_ref, acc_s):
    i = pl.program_id(0)
    z = z_ref[...]
    z2 = z2_ref[...]
    lcol = jax.lax.broadcasted_iota(jnp.int32, (T, K), 1)

    minv = jnp.full((T, 1), jnp.inf, jnp.float32)
    mini = jnp.zeros((T, 1), jnp.int32)
    for k in range(NK):
        m2 = jax.lax.dot_general(
            z, emb2_ref[k * K:(k + 1) * K, :],
            dimension_numbers=(((1,), (1,)), ((), ())),
            preferred_element_type=jnp.float32,
            precision=_DOT_PREC,
        )
        d = (z2 + e2_ref[:, k * K:(k + 1) * K]) - m2
        rowmin = jnp.min(d, axis=1, keepdims=True)
        rowidx = jnp.min(jnp.where(d == rowmin, lcol, K),
                         axis=1, keepdims=True) + k * K
        upd = rowmin < minv
        mini = jnp.where(upd, rowidx, mini)
        minv = jnp.where(upd, rowmin, minv)
    idx_ref[...] = mini

    for k in range(NK):
        oh_ref[:, k * K:(k + 1) * K] = (
            lcol == (mini - k * K)).astype(jnp.float32)

    sq = jnp.sum(minv)

    @pl.when(i == 0)
    def _init():
        acc_s[0] = sq

    @pl.when(i > 0)
    def _acc():
        acc_s[0] += sq

    @pl.when(i == NT - 1)
    def _finalize():
        mse = acc_s[0] / jnp.float32(N_TOK * C_DIM)
        loss_ref[...] = jnp.full((1, 1), mse + BETA * mse, jnp.float32)


def _tc_call(z2, e2, z_flat, emb2, interpret=False):
    return pl.pallas_call(
        _tc_body,
        grid=(NT,),
        in_specs=[
            pl.BlockSpec((T, 1), lambda i: (i, 0)),
            pl.BlockSpec((1, N_E), lambda i: (0, 0)),
            pl.BlockSpec((T, C_DIM), lambda i: (i, 0)),
            pl.BlockSpec((N_E, C_DIM), lambda i: (0, 0)),
        ],
        out_specs=[
            pl.BlockSpec((T, N_E), lambda i: (i, 0)),
            pl.BlockSpec((T, 1), lambda i: (i, 0)),
            pl.BlockSpec((1, 1), lambda i: (0, 0)),
        ],
        out_shape=[
            jax.ShapeDtypeStruct((N_TOK, N_E), jnp.float32),
            jax.ShapeDtypeStruct((N_TOK, 1), jnp.int32),
            jax.ShapeDtypeStruct((1, 1), jnp.float32),
        ],
        scratch_shapes=[
            pltpu.SMEM((1,), jnp.float32),
        ],
        interpret=interpret,
    )(z2, e2, z_flat, emb2)


_SC_INFO = plsc.get_sparse_core_info()
_NC = _SC_INFO.num_cores
_NS = _SC_INFO.num_subcores
_NW = _NC * _NS
_BPW = N_TOK // _NW
_HPW = N_E // _NS


def _sc_body(emb_hbm, idx_hbm, zq_hbm, hist_hbm,
             idx_v, rows_v, ones_v, chunk_v, hist_sh, sem):
    c = lax.axis_index("c")
    s = lax.axis_index("s")
    wid = s * _NC + c
    base = wid * _BPW

    pltpu.sync_copy(idx_hbm.at[pl.ds(base, _BPW)], idx_v)
    pltpu.async_copy(emb_hbm.at[idx_v], rows_v, sem).wait()
    pltpu.sync_copy(rows_v, zq_hbm.at[pl.ds(base, _BPW)])

    for t in range(_HPW // 16):
        chunk_v[pl.ds(t * 16, 16)] = jnp.zeros((16,), jnp.float32)
    for t in range(_BPW // 16):
        ones_v[pl.ds(t * 16, 16)] = jnp.ones((16,), jnp.float32)
    pltpu.sync_copy(chunk_v, hist_sh.at[pl.ds(s * _HPW, _HPW)])
    plsc.subcore_barrier()
    pltpu.sync_copy(ones_v, hist_sh.at[idx_v], add=True)
    plsc.subcore_barrier()
    pltpu.sync_copy(hist_sh.at[pl.ds(s * _HPW, _HPW)], chunk_v)
    pltpu.sync_copy(chunk_v, hist_hbm.at[c, pl.ds(s * _HPW, _HPW)])


@functools.partial(
    pl.kernel,
    mesh=plsc.VectorSubcoreMesh(core_axis_name="c", subcore_axis_name="s"),
    out_type=[
        jax.ShapeDtypeStruct((N_TOK, 128), jnp.float32),
        jax.ShapeDtypeStruct((_NC, N_E), jnp.float32),
    ],
    scratch_types=[
        pltpu.VMEM((_BPW,), jnp.int32),
        pltpu.VMEM((_BPW, 128), jnp.float32),
        pltpu.VMEM((_BPW,), jnp.float32),
        pltpu.VMEM((_HPW,), jnp.float32),
        pltpu.VMEM_SHARED((N_E,), jnp.float32),
        pltpu.SemaphoreType.DMA,
    ],
)
def _sc_call(emb_hbm, idx_hbm, zq_hbm, hist_hbm,
             idx_v, rows_v, ones_v, chunk_v, hist_sh, sem):
    _sc_body(emb_hbm, idx_hbm, zq_hbm, hist_hbm,
             idx_v, rows_v, ones_v, chunk_v, hist_sh, sem)


def _perp_body(hist_ref, perp_ref):
    h = hist_ref[...]
    e_mean = (h[0:1, :] + h[1:2, :]) / jnp.float32(N_TOK)
    ent = -jnp.sum(e_mean * jnp.log(e_mean + 1e-10))
    perp_ref[...] = jnp.full((1, 1), jnp.exp(ent), jnp.float32)


def _perp_call(hist, interpret=False):
    return pl.pallas_call(
        _perp_body,
        grid=(1,),
        in_specs=[pl.BlockSpec((_NC, N_E), lambda i: (0, 0))],
        out_specs=pl.BlockSpec((1, 1), lambda i: (0, 0)),
        out_shape=jax.ShapeDtypeStruct((1, 1), jnp.float32),
        interpret=interpret,
    )(hist)


def kernel(z, emb_weight, *, interpret=False):
    zp = jnp.transpose(z, (0, 2, 3, 1))
    z_flat = zp.reshape(-1, C_DIM)
    z2 = jnp.sum(z_flat ** 2, axis=1, keepdims=True)
    e2 = jnp.sum(emb_weight ** 2, axis=1)[None, :]
    emb2 = emb_weight * 2.0

    min_encodings, min_idx, loss = _tc_call(
        z2, e2, z_flat, emb2, interpret=interpret)

    emb_pad = jnp.pad(emb_weight, ((0, 0), (0, 128 - C_DIM)))
    zq_pad, hist = _sc_call(emb_pad, min_idx.reshape(-1))
    zq = zq_pad[:, :C_DIM]

    perp = _perp_call(hist, interpret=interpret)

    z_q_out = jnp.transpose(zq.reshape(zp.shape), (0, 3, 1, 2))
    return (loss[0, 0], z_q_out, perp[0, 0], min_encodings, min_idx)


# speedup vs baseline: 1.2398x; 1.0017x over previous
import functools

import jax
import jax.numpy as jnp
from jax import lax
from jax.experimental import pallas as pl
from jax.experimental.pallas import tpu as pltpu
from jax.experimental.pallas import tpu_sc as plsc

N_E = 8192
C_DIM = 32
BETA = 0.25
N_TOK = 4096

T = 512
K = 1024
NT = N_TOK // T
NK = N_E // K

_DOT_PREC = jax.lax.Precision.DEFAULT


NH = N_E // 2


def _tc_body(z2_ref, e2_ref, z_ref, emb2_ref,
             oh_ref, idx_ref, loss_ref, acc_s):
    i = pl.program_id(0)
    z = z_ref[...]
    z2 = z2_ref[...]
    lcol = jax.lax.broadcasted_iota(jnp.int32, (T, K), 1)

    minv = jnp.full((T, 1), jnp.inf, jnp.float32)
    mini = jnp.zeros((T, 1), jnp.int32)
    for k in range(NK):
        m2 = jax.lax.dot_general(
            z, emb2_ref[k * K:(k + 1) * K, :],
            dimension_numbers=(((1,), (1,)), ((), ())),
            preferred_element_type=jnp.float32,
            precision=_DOT_PREC,
        )
        d = (z2 + e2_ref[:, k * K:(k + 1) * K]) - m2
        rowmin = jnp.min(d, axis=1, keepdims=True)
        rowidx = jnp.min(jnp.where(d == rowmin, lcol, K),
                         axis=1, keepdims=True) + k * K
        upd = rowmin < minv
        mini = jnp.where(upd, rowidx, mini)
        minv = jnp.where(upd, rowmin, minv)
    idx_ref[...] = mini

    for k in range(NK):
        oh_ref[:, k * K:(k + 1) * K] = (
            lcol == (mini - k * K)).astype(jnp.float32)

    sq = jnp.sum(minv)

    @pl.when(i == 0)
    def _init():
        acc_s[0] = sq

    @pl.when(i > 0)
    def _acc():
        acc_s[0] += sq

    @pl.when(i == NT - 1)
    def _finalize():
        mse = acc_s[0] / jnp.float32(N_TOK * C_DIM)
        loss_ref[...] = jnp.full((1, 1), mse + BETA * mse, jnp.float32)


def _tc_call(z2, e2, z_flat, emb2, interpret=False):
    return pl.pallas_call(
        _tc_body,
        grid=(NT,),
        in_specs=[
            pl.BlockSpec((T, 1), lambda i: (i, 0)),
            pl.BlockSpec((1, N_E), lambda i: (0, 0)),
            pl.BlockSpec((T, C_DIM), lambda i: (i, 0)),
            pl.BlockSpec((N_E, C_DIM), lambda i: (0, 0)),
        ],
        out_specs=[
            pl.BlockSpec((T, N_E), lambda i: (i, 0)),
            pl.BlockSpec((T, 1), lambda i: (i, 0)),
            pl.BlockSpec((1, 1), lambda i: (0, 0)),
        ],
        out_shape=[
            jax.ShapeDtypeStruct((N_TOK, N_E), jnp.float32),
            jax.ShapeDtypeStruct((N_TOK, 1), jnp.int32),
            jax.ShapeDtypeStruct((1, 1), jnp.float32),
        ],
        scratch_shapes=[
            pltpu.SMEM((1,), jnp.float32),
        ],
        interpret=interpret,
    )(z2, e2, z_flat, emb2)


_NC = 2
_NS = 16
_NW = _NC * _NS
_BPW = N_TOK // _NW
_HPW = N_E // _NS


def _sc_body(emb_hbm, idx_hbm, zq_hbm, hist_hbm,
             idx_v, rows_v, ones_v, chunk_v, hist_sh, sem):
    c = lax.axis_index("c")
    s = lax.axis_index("s")
    wid = s * _NC + c
    base = wid * _BPW

    pltpu.sync_copy(idx_hbm.at[pl.ds(base, _BPW)], idx_v)
    pltpu.async_copy(emb_hbm.at[idx_v], rows_v, sem).wait()
    pltpu.sync_copy(rows_v, zq_hbm.at[pl.ds(base, _BPW)])

    for t in range(_HPW // 16):
        chunk_v[pl.ds(t * 16, 16)] = jnp.zeros((16,), jnp.float32)
    for t in range(_BPW // 16):
        ones_v[pl.ds(t * 16, 16)] = jnp.ones((16,), jnp.float32)
    pltpu.sync_copy(chunk_v, hist_sh.at[pl.ds(s * _HPW, _HPW)])
    plsc.subcore_barrier()
    pltpu.sync_copy(ones_v, hist_sh.at[idx_v], add=True)
    plsc.subcore_barrier()
    pltpu.sync_copy(hist_sh.at[pl.ds(s * _HPW, _HPW)], chunk_v)
    pltpu.sync_copy(chunk_v, hist_hbm.at[c, pl.ds(s * _HPW, _HPW)])


@functools.partial(
    pl.kernel,
    mesh=plsc.VectorSubcoreMesh(core_axis_name="c", subcore_axis_name="s"),
    out_type=[
        jax.ShapeDtypeStruct((N_TOK, 128), jnp.float32),
        jax.ShapeDtypeStruct((_NC, N_E), jnp.float32),
    ],
    scratch_types=[
        pltpu.VMEM((_BPW,), jnp.int32),
        pltpu.VMEM((_BPW, 128), jnp.float32),
        pltpu.VMEM((_BPW,), jnp.float32),
        pltpu.VMEM((_HPW,), jnp.float32),
        pltpu.VMEM_SHARED((N_E,), jnp.float32),
        pltpu.SemaphoreType.DMA,
    ],
)
def _sc_call(emb_hbm, idx_hbm, zq_hbm, hist_hbm,
             idx_v, rows_v, ones_v, chunk_v, hist_sh, sem):
    _sc_body(emb_hbm, idx_hbm, zq_hbm, hist_hbm,
             idx_v, rows_v, ones_v, chunk_v, hist_sh, sem)


def _perp_body(hist_ref, perp_ref):
    h = hist_ref[...]
    e_mean = (h[0:1, :] + h[1:2, :]) / jnp.float32(N_TOK)
    ent = -jnp.sum(e_mean * jnp.log(e_mean + 1e-10))
    perp_ref[...] = jnp.full((1, 1), jnp.exp(ent), jnp.float32)


def _perp_call(hist, interpret=False):
    return pl.pallas_call(
        _perp_body,
        grid=(1,),
        in_specs=[pl.BlockSpec((_NC, N_E), lambda i: (0, 0))],
        out_specs=pl.BlockSpec((1, 1), lambda i: (0, 0)),
        out_shape=jax.ShapeDtypeStruct((1, 1), jnp.float32),
        interpret=interpret,
    )(hist)


def kernel(z, emb_weight, *, interpret=False):
    zp = jnp.transpose(z, (0, 2, 3, 1))
    z_flat = zp.reshape(-1, C_DIM)
    z2 = jnp.sum(z_flat ** 2, axis=1, keepdims=True)
    e2 = jnp.sum(emb_weight ** 2, axis=1)[None, :]
    emb2 = emb_weight * 2.0

    min_encodings, min_idx, loss = _tc_call(
        z2, e2, z_flat, emb2, interpret=interpret)

    emb_pad = jnp.pad(emb_weight, ((0, 0), (0, 128 - C_DIM)))
    zq_pad, hist = _sc_call(emb_pad, min_idx.reshape(-1))
    zq = zq_pad[:, :C_DIM]

    perp = _perp_call(hist, interpret=interpret)

    z_q_out = jnp.transpose(zq.reshape(zp.shape), (0, 3, 1, 2))
    return (loss[0, 0], z_q_out, perp[0, 0], min_encodings, min_idx)
